# baseline (speedup 1.0000x reference)
import jax
import jax.numpy as jnp
from jax.experimental import pallas as pl


def _tc_kernel(idx_ref, emb_ref, wt_ref, b_ref, out_ref):
    rows = idx_ref.shape[0]
    vocab = emb_ref.shape[0]
    iota = jax.lax.broadcasted_iota(jnp.int32, (rows, vocab), 1)
    oh = (idx_ref[...] == iota).astype(jnp.float32)
    e = jnp.dot(oh, emb_ref[...], preferred_element_type=jnp.float32)
    y = jnp.dot(e, wt_ref[...], preferred_element_type=jnp.float32) + b_ref[...]
    bb = out_ref.shape[0]
    out_ref[...] = y.reshape(bb, out_ref.shape[1], out_ref.shape[2])


def kernel(input_seq, emb_table, W, b):
    batch, seq = input_seq.shape
    vocab, dim = emb_table.shape
    idx2 = input_seq.reshape(batch * seq, 1).astype(jnp.int32)
    wt = W.T
    b2 = b.reshape(1, vocab)
    bb = 8
    grid = (batch // bb,)
    return pl.pallas_call(
        _tc_kernel,
        grid=grid,
        in_specs=[
            pl.BlockSpec((bb * seq, 1), lambda i: (i, 0)),
            pl.BlockSpec((vocab, dim), lambda i: (0, 0)),
            pl.BlockSpec((dim, vocab), lambda i: (0, 0)),
            pl.BlockSpec((1, vocab), lambda i: (0, 0)),
        ],
        out_specs=pl.BlockSpec((bb, seq, vocab), lambda i: (i, 0, 0)),
        out_shape=jax.ShapeDtypeStruct((batch, seq, vocab), jnp.float32),
    )(idx2, emb_table, wt, b2)
